# baseline (speedup 1.0000x reference)
import jax
import jax.numpy as jnp
from jax.experimental import pallas as pl
from jax.experimental.pallas import tpu as pltpu

HIDDEN = 1024
FMIN = 0.1
FMAX = 0.5

B = 128
F = 32768
CHUNK_ROWS = (32, 32, 32, 24, 8)
CHUNK_BASE = tuple(sum(CHUNK_ROWS[:i]) for i in range(len(CHUNK_ROWS)))
NCHUNK = len(CHUNK_ROWS)


def _psd_peak_body(x_hbm, out_ref, *scratch):
    bufs, sems = scratch[:NCHUNK], scratch[NCHUNK]

    copies = [
        pltpu.make_async_copy(
            x_hbm.at[pl.ds(CHUNK_BASE[c], CHUNK_ROWS[c]), :], bufs[c], sems.at[c]
        )
        for c in range(NCHUNK)
    ]
    for cp in copies:
        cp.start()

    for c in range(NCHUNK):
        copies[c].wait()
        rows = CHUNK_ROWS[c]

        blk = bufs[c][...]
        bmax = jnp.max(blk, axis=1, keepdims=True)
        iota = jax.lax.broadcasted_iota(jnp.int32, (rows, F), 1)
        cand = jnp.where(blk == bmax, iota, F)
        peak = jnp.min(cand, axis=1, keepdims=True)

        idxf = peak.astype(jnp.float32)
        freq = FMIN + (FMAX - FMIN) * idxf / (F - 1)
        rr = freq * 60.0
        out_ref[pl.ds(CHUNK_BASE[c], rows), :] = jnp.broadcast_to(rr, (rows, HIDDEN))


_psd_peak = pl.pallas_call(
    _psd_peak_body,
    in_specs=[pl.BlockSpec(memory_space=pl.ANY)],
    out_specs=pl.BlockSpec((B, HIDDEN), memory_space=pltpu.MemorySpace.VMEM),
    out_shape=jax.ShapeDtypeStruct((B, HIDDEN), jnp.float32),
    scratch_shapes=[pltpu.VMEM((r, F), jnp.float32) for r in CHUNK_ROWS]
    + [pltpu.SemaphoreType.DMA((NCHUNK,))],
)


def kernel(x):
    return _psd_peak(x)
